# initial kernel scaffold (unmeasured)
import jax
import jax.numpy as jnp
from jax import lax
from jax.experimental import pallas as pl
from jax.experimental.pallas import tpu as pltpu


def kernel(
    x,
):
    def body(*refs):
        pass

    out_shape = jax.ShapeDtypeStruct(..., jnp.float32)
    return pl.pallas_call(body, out_shape=out_shape)(...)


# baseline (device time: 159862 ns/iter reference)
import jax
import jax.numpy as jnp
from jax import lax
from jax.experimental import pallas as pl
from jax.experimental.pallas import tpu as pltpu

K = 32
CHUNK = 1024
NEG = float(jnp.finfo(jnp.float32).min)


def _topk_desc(block, k, exact):
    rows, w = block.shape
    cur = block
    outs = []
    if exact:
        iota = lax.broadcasted_iota(jnp.int32, (rows, w), 1)
    for _ in range(k):
        m = jnp.max(cur, axis=1, keepdims=True)
        outs.append(m)
        if exact:
            is_m = cur == m
            first = jnp.min(jnp.where(is_m, iota, w), axis=1, keepdims=True)
            cur = jnp.where(iota == first, NEG, cur)
        else:
            cur = jnp.where(cur == m, NEG, cur)
    return jnp.concatenate(outs, axis=1)


def _stage1_body(x_ref, out_ref):
    out_ref[0, :, :] = _topk_desc(x_ref[...], K, exact=False)


def _stage2_body(cand_ref, out_ref, send_ref, recv_ref, send_sem, recv_sem):
    my_x = lax.axis_index("x")
    my_y = lax.axis_index("y")
    my_z = lax.axis_index("z")
    partner = (1 - my_x, my_y, my_z)

    n_chunks = cand_ref.shape[0]
    cands = jnp.concatenate(
        [cand_ref[j, :, :] for j in range(n_chunks)], axis=1
    )
    mine = _topk_desc(cands, K, exact=True)
    send_ref[...] = mine

    bsem = pltpu.get_barrier_semaphore()
    pl.semaphore_signal(
        bsem, inc=1, device_id=partner, device_id_type=pl.DeviceIdType.MESH
    )
    pl.semaphore_wait(bsem, 1)

    rdma = pltpu.make_async_remote_copy(
        src_ref=send_ref,
        dst_ref=recv_ref,
        send_sem=send_sem,
        recv_sem=recv_sem,
        device_id=partner,
        device_id_type=pl.DeviceIdType.MESH,
    )
    rdma.start()
    rdma.wait()

    both = jnp.concatenate([mine, recv_ref[...]], axis=1)
    out_ref[...] = _topk_desc(both, K, exact=True)


def kernel(x):
    m, n = x.shape
    n_chunks = n // CHUNK

    cands = pl.pallas_call(
        _stage1_body,
        grid=(n_chunks,),
        in_specs=[pl.BlockSpec((m, CHUNK), lambda j: (0, j))],
        out_specs=pl.BlockSpec((1, m, K), lambda j: (j, 0, 0)),
        out_shape=jax.ShapeDtypeStruct((n_chunks, m, K), jnp.float32),
    )(x)

    return pl.pallas_call(
        _stage2_body,
        out_shape=jax.ShapeDtypeStruct((m, K), jnp.float32),
        in_specs=[pl.BlockSpec(memory_space=pltpu.VMEM)],
        out_specs=pl.BlockSpec(memory_space=pltpu.VMEM),
        scratch_shapes=[
            pltpu.VMEM((m, K), jnp.float32),
            pltpu.VMEM((m, K), jnp.float32),
            pltpu.SemaphoreType.DMA,
            pltpu.SemaphoreType.DMA,
        ],
        compiler_params=pltpu.CompilerParams(collective_id=0),
    )(cands)


# device time: 97675 ns/iter; 1.6367x vs baseline; 1.6367x over previous
import jax
import jax.numpy as jnp
from jax import lax
from jax.experimental import pallas as pl
from jax.experimental.pallas import tpu as pltpu

K = 32
CUTOFF = 1024
NEG = float(jnp.finfo(jnp.float32).min)


def _topk_desc(block, k, exact):
    rows, w = block.shape
    cur = block
    outs = []
    if exact:
        iota = lax.broadcasted_iota(jnp.int32, (rows, w), 1)
    for _ in range(k):
        m = jnp.max(cur, axis=1, keepdims=True)
        outs.append(m)
        if exact:
            is_m = cur == m
            first = jnp.min(jnp.where(is_m, iota, w), axis=1, keepdims=True)
            cur = jnp.where(iota == first, NEG, cur)
        else:
            cur = jnp.where(cur == m, NEG, cur)
    return jnp.concatenate(outs, axis=1)


def _tournament_candidates(x, k, cutoff):
    w = x.shape[1]
    if k >= w:
        return [x]
    if w <= cutoff:
        return [_topk_desc(x, k, exact=False)]
    h = w // 2
    a, b = x[:, :h], x[:, h:]
    return _tournament_candidates(jnp.maximum(a, b), k, cutoff) + (
        _tournament_candidates(jnp.minimum(a, b), max(k // 2, 1), cutoff)
    )


def _stage1_body(x_ref, out_ref):
    rows = x_ref.shape[0]
    cands = _tournament_candidates(x_ref[...], K, CUTOFF)
    c = jnp.concatenate(cands, axis=1)
    pad = out_ref.shape[1] - c.shape[1]
    out_ref[...] = jnp.concatenate(
        [c, jnp.full((rows, pad), NEG, jnp.float32)], axis=1
    )


def _stage2_body(cand_ref, out_ref, send_ref, recv_ref, send_sem, recv_sem):
    my_x = lax.axis_index("x")
    my_y = lax.axis_index("y")
    my_z = lax.axis_index("z")
    partner = (1 - my_x, my_y, my_z)

    mine = _topk_desc(cand_ref[...], K, exact=True)
    send_ref[...] = mine

    bsem = pltpu.get_barrier_semaphore()
    pl.semaphore_signal(
        bsem, inc=1, device_id=partner, device_id_type=pl.DeviceIdType.MESH
    )
    pl.semaphore_wait(bsem, 1)

    rdma = pltpu.make_async_remote_copy(
        src_ref=send_ref,
        dst_ref=recv_ref,
        send_sem=send_sem,
        recv_sem=recv_sem,
        device_id=partner,
        device_id_type=pl.DeviceIdType.MESH,
    )
    rdma.start()
    rdma.wait()

    both = jnp.concatenate([mine, recv_ref[...]], axis=1)
    out_ref[...] = _topk_desc(both, K, exact=True)


def kernel(x):
    m, n = x.shape

    RB = 128
    cands = pl.pallas_call(
        _stage1_body,
        grid=(m // RB,),
        in_specs=[pl.BlockSpec((RB, n), lambda i: (i, 0))],
        out_specs=pl.BlockSpec((RB, 128), lambda i: (i, 0)),
        out_shape=jax.ShapeDtypeStruct((m, 128), jnp.float32),
    )(x)

    return pl.pallas_call(
        _stage2_body,
        out_shape=jax.ShapeDtypeStruct((m, K), jnp.float32),
        in_specs=[pl.BlockSpec(memory_space=pltpu.VMEM)],
        out_specs=pl.BlockSpec(memory_space=pltpu.VMEM),
        scratch_shapes=[
            pltpu.VMEM((m, K), jnp.float32),
            pltpu.VMEM((m, K), jnp.float32),
            pltpu.SemaphoreType.DMA,
            pltpu.SemaphoreType.DMA,
        ],
        compiler_params=pltpu.CompilerParams(collective_id=0),
    )(cands)


# device time: 38393 ns/iter; 4.1638x vs baseline; 2.5441x over previous
import jax
import jax.numpy as jnp
from jax import lax
from jax.experimental import pallas as pl
from jax.experimental.pallas import tpu as pltpu

K = 32
N_SLICES = 8
PACK = 4
NEG = float(jnp.finfo(jnp.float32).min)


def _topk_desc(block, k):
    cur = block
    outs = []
    for _ in range(k):
        m = jnp.max(cur, axis=1, keepdims=True)
        outs.append(m)
        cur = jnp.where(cur == m, NEG, cur)
    return jnp.concatenate(outs, axis=1)


def _roll_seg(c, s):
    s = s % K
    pieces = []
    for j in range(c.shape[1] // K):
        seg = c[:, j * K : (j + 1) * K]
        pieces.extend([seg[:, s:], seg[:, :s]])
    return jnp.concatenate(pieces, axis=1)


def _merge_top32_packed(a, b):
    rows, w = a.shape
    lane = lax.broadcasted_iota(jnp.int32, (rows, w), 1)
    b_rev = b
    for s in (16, 8, 4, 2, 1):
        b_rev = jnp.where(
            lane % (2 * s) < s, _roll_seg(b_rev, s), _roll_seg(b_rev, -s)
        )
    c = jnp.maximum(a, b_rev)
    for s in (16, 8, 4, 2, 1):
        hi = jnp.maximum(c, _roll_seg(c, s))
        lo = jnp.minimum(c, _roll_seg(c, -s))
        c = jnp.where(lane % (2 * s) < s, hi, lo)
    return c


def _body(
    x_hbm,
    out_ref,
    xs_ref,
    merged_ref,
    send_ref,
    recv_ref,
    copy_sems,
    send_sems,
    recv_sems,
):
    my_x = lax.axis_index("x")
    my_y = lax.axis_index("y")
    my_z = lax.axis_index("z")
    partners = [
        (my_x, 1 - my_y, my_z),
        (my_x, my_y, my_z ^ 1),
        (my_x, my_y, my_z ^ 2),
        (1 - my_x, my_y, my_z),
    ]
    rows_p = merged_ref.shape[0]

    bsem = pltpu.get_barrier_semaphore()
    for p in partners:
        pl.semaphore_signal(
            bsem, inc=1, device_id=p, device_id_type=pl.DeviceIdType.MESH
        )
    pl.semaphore_wait(bsem, len(partners))

    n_slice = xs_ref.shape[2]
    r = my_y * 4 + my_z
    for j in range(PACK):
        pltpu.make_async_copy(
            x_hbm.at[pl.ds(j * rows_p, rows_p), pl.ds(r * n_slice, n_slice)],
            xs_ref.at[j],
            copy_sems.at[j],
        ).start()
    for j in range(PACK):
        pltpu.make_async_copy(
            x_hbm.at[pl.ds(j * rows_p, rows_p), pl.ds(r * n_slice, n_slice)],
            xs_ref.at[j],
            copy_sems.at[j],
        ).wait()

    curs = [xs_ref[j] for j in range(PACK)]
    outs = [[] for _ in range(PACK)]
    for _ in range(K):
        for j in range(PACK):
            m = jnp.max(curs[j], axis=1, keepdims=True)
            outs[j].append(m)
            curs[j] = jnp.where(curs[j] == m, NEG, curs[j])
    packed = jnp.concatenate(
        [jnp.concatenate(outs[j], axis=1) for j in range(PACK)], axis=1
    )
    rows_h = rows_p // 2
    send_ref[0] = packed[:rows_h, :]
    send_ref[1] = packed[rows_h:, :]

    n_rounds = len(partners)
    rdmas = [
        pltpu.make_async_remote_copy(
            src_ref=send_ref.at[2 * rd + h],
            dst_ref=recv_ref.at[2 * rd + h],
            send_sem=send_sems.at[2 * rd + h],
            recv_sem=recv_sems.at[2 * rd + h],
            device_id=p,
            device_id_type=pl.DeviceIdType.MESH,
        )
        for rd, p in enumerate(partners)
        for h in range(2)
    ]
    rdmas[0].start()
    rdmas[1].start()
    for rd in range(n_rounds):
        for h in range(2):
            s = 2 * rd + h
            rdmas[s].wait()
            merged = _merge_top32_packed(send_ref[s], recv_ref[s])
            if rd + 1 < n_rounds:
                send_ref[s + 2] = merged
                rdmas[s + 2].start()
            else:
                merged_ref[pl.ds(h * rows_h, rows_h), :] = merged

    for j in range(PACK):
        out_ref[pl.ds(j * rows_p, rows_p), :] = merged_ref[
            :, j * K : (j + 1) * K
        ]


def kernel(x):
    m, n = x.shape
    n_slice = n // N_SLICES
    n_rounds = 4
    m_p = m // PACK

    return pl.pallas_call(
        _body,
        out_shape=jax.ShapeDtypeStruct((m, K), jnp.float32),
        in_specs=[pl.BlockSpec(memory_space=pltpu.MemorySpace.HBM)],
        out_specs=pl.BlockSpec(memory_space=pltpu.VMEM),
        scratch_shapes=[
            pltpu.VMEM((PACK, m_p, n_slice), jnp.float32),
            pltpu.VMEM((m_p, 128), jnp.float32),
            pltpu.VMEM((2 * n_rounds, m_p // 2, 128), jnp.float32),
            pltpu.VMEM((2 * n_rounds, m_p // 2, 128), jnp.float32),
            pltpu.SemaphoreType.DMA((PACK,)),
            pltpu.SemaphoreType.DMA((2 * n_rounds,)),
            pltpu.SemaphoreType.DMA((2 * n_rounds,)),
        ],
        compiler_params=pltpu.CompilerParams(collective_id=0),
    )(x)
